# baseline (speedup 1.0000x reference)
import jax
import jax.numpy as jnp
from jax import lax
from jax.experimental import pallas as pl
from jax.experimental.pallas import tpu as pltpu
from jax.experimental.pallas import tpu_sc as plsc

_NUM_CLASSES = 1000
_BATCH = 512
_LANES = 16


def _body(xt_hbm, out_hbm, xv, ov, pbuf, p8, s16, shared, sem):
    c = lax.axis_index("c")
    s = lax.axis_index("s")
    g = s // 8
    h = s - g * 8
    bj = 2 * c + g
    b0 = bj * 128

    cp = pltpu.async_copy(
        xt_hbm.at[:, h, pl.ds(0, 8), pl.ds(b0, 128)], xv, sem
    )

    zeros16 = jnp.zeros((_LANES,), jnp.float32)

    def _zrow(row, carry):
        def _zchunk(j, c2):
            off = pl.multiple_of(j * _LANES, _LANES)
            ov[row, pl.ds(off, _LANES)] = zeros16
            return c2

        lax.fori_loop(0, _NUM_CLASSES // _LANES, _zchunk, 0)
        ov[row, pl.ds(_NUM_CLASSES - _LANES, _LANES)] = zeros16
        return carry

    lax.fori_loop(0, 16, _zrow, 0)

    cp.wait()

    def _red_k(k, carry):
        koff = pl.multiple_of(k * _LANES, _LANES)

        def _red_t(t, acc):
            ch = t // 8
            w = t - ch * 8
            return acc + xv[ch, w, pl.ds(koff, _LANES)]

        acc = lax.fori_loop(0, 24, _red_t, zeros16)
        pbuf[pl.ds(koff, _LANES)] = acc
        return carry

    lax.fori_loop(0, 8, _red_k, 0)

    pltpu.sync_copy(pbuf, shared.at[s])
    plsc.subcore_barrier()
    pltpu.sync_copy(shared.at[pl.ds(g * 8, 8), :], p8)

    hoff = pl.multiple_of(h * _LANES, _LANES)

    def _comb(i, tot):
        return tot + p8[i, pl.ds(hoff, _LANES)]

    total = lax.fori_loop(0, 8, _comb, zeros16)

    mean = total / 192.0
    scaled = mean * 10.0
    pred = scaled.astype(jnp.int32)
    rem = lax.rem(pred, _NUM_CLASSES)
    pred = jnp.where(rem < 0, rem + _NUM_CLASSES, rem)

    lanes = lax.iota(jnp.int32, _LANES)
    plsc.store_scatter(ov, [lanes, pred], jnp.full((_LANES,), 10.0, jnp.float32))

    base = b0 + h * _LANES
    pltpu.sync_copy(ov, out_hbm.at[pl.ds(base, _LANES), :])
    del s16


@jax.jit
def kernel(x):
    xt = jnp.transpose(x, (1, 2, 3, 0))
    return pl.kernel(
        _body,
        out_type=jax.ShapeDtypeStruct((_BATCH, _NUM_CLASSES), jnp.float32),
        mesh=plsc.VectorSubcoreMesh(core_axis_name="c", subcore_axis_name="s"),
        scratch_types=[
            pltpu.VMEM((3, 8, 128), jnp.float32),
            pltpu.VMEM((_LANES, _NUM_CLASSES), jnp.float32),
            pltpu.VMEM((128,), jnp.float32),
            pltpu.VMEM((8, 128), jnp.float32),
            pltpu.VMEM((_LANES,), jnp.float32),
            pltpu.VMEM_SHARED((16, 128), jnp.float32),
            pltpu.SemaphoreType.DMA,
        ],
        compiler_params=pltpu.CompilerParams(needs_layout_passes=False),
    )(xt)


# speedup vs baseline: 1.2454x; 1.2454x over previous
import jax
import jax.numpy as jnp
from jax import lax
from jax.experimental import pallas as pl
from jax.experimental.pallas import tpu as pltpu
from jax.experimental.pallas import tpu_sc as plsc

_NUM_CLASSES = 1000
_BATCH = 512
_LANES = 16


def _body(xt_hbm, out_hbm, xv, ov, pbuf, p8, shared, sem):
    c = lax.axis_index("c")
    s = lax.axis_index("s")
    g = s // 8
    q = s - g * 8
    bj = 2 * c + g
    b0 = bj * 128

    cp = pltpu.async_copy(
        xt_hbm.at[:, q, pl.ds(0, 8), pl.ds(b0, 128)], xv, sem
    )

    zeros16 = jnp.zeros((_LANES,), jnp.float32)

    def _zrow(row, carry):
        for j in range(8):
            ov[row, pl.ds(j * _LANES, _LANES)] = zeros16
        return carry

    lax.fori_loop(0, 128, _zrow, 0)

    cp.wait()

    def _red_k(k, carry):
        koff = pl.multiple_of(k * _LANES, _LANES)

        def _red_t(t, acc):
            ch = t // 8
            w = t - ch * 8
            return acc + xv[ch, w, pl.ds(koff, _LANES)]

        acc = lax.fori_loop(0, 24, _red_t, zeros16)
        pbuf[pl.ds(koff, _LANES)] = acc
        return carry

    lax.fori_loop(0, 8, _red_k, 0)

    pltpu.sync_copy(pbuf, shared.at[s])
    plsc.subcore_barrier()
    pltpu.sync_copy(shared.at[pl.ds(g * 8, 8), :], p8)

    rowbase = q * 128
    lanes = lax.iota(jnp.int32, _LANES)
    tens = jnp.full((_LANES,), 10.0, jnp.float32)

    def _scat_k(k, carry):
        koff = pl.multiple_of(k * _LANES, _LANES)

        def _comb(i, tot):
            return tot + p8[i, pl.ds(koff, _LANES)]

        total = lax.fori_loop(0, 8, _comb, zeros16)
        mean = total / 192.0
        scaled = mean * 10.0
        pred = scaled.astype(jnp.int32)
        rem = lax.rem(pred, _NUM_CLASSES)
        pred = jnp.where(rem < 0, rem + _NUM_CLASSES, rem)

        r = pred - rowbase
        mask = (r >= 0) & (r < 128)
        r = jnp.where(mask, r, 0)
        plsc.store_scatter(ov, [r, lanes + k * _LANES], tens, mask=mask)
        return carry

    lax.fori_loop(0, 8, _scat_k, 0)

    @pl.when(q < 7)
    def _full():
        pltpu.sync_copy(ov, out_hbm.at[pl.ds(rowbase, 128), pl.ds(b0, 128)])

    @pl.when(q == 7)
    def _tail():
        pltpu.sync_copy(
            ov.at[pl.ds(0, _NUM_CLASSES - 7 * 128), :],
            out_hbm.at[pl.ds(rowbase, _NUM_CLASSES - 7 * 128), pl.ds(b0, 128)],
        )


@jax.jit
def kernel(x):
    xt = jnp.transpose(x, (1, 2, 3, 0))
    out_t = pl.kernel(
        _body,
        out_type=jax.ShapeDtypeStruct((_NUM_CLASSES, _BATCH), jnp.float32),
        mesh=plsc.VectorSubcoreMesh(core_axis_name="c", subcore_axis_name="s"),
        scratch_types=[
            pltpu.VMEM((3, 8, 128), jnp.float32),
            pltpu.VMEM((128, 128), jnp.float32),
            pltpu.VMEM((128,), jnp.float32),
            pltpu.VMEM((8, 128), jnp.float32),
            pltpu.VMEM_SHARED((16, 128), jnp.float32),
            pltpu.SemaphoreType.DMA,
        ],
        compiler_params=pltpu.CompilerParams(needs_layout_passes=False),
    )(xt)
    return out_t.T


# speedup vs baseline: 1.2526x; 1.0058x over previous
import jax
import jax.numpy as jnp
from jax import lax
from jax.experimental import pallas as pl
from jax.experimental.pallas import tpu as pltpu
from jax.experimental.pallas import tpu_sc as plsc

_NUM_CLASSES = 1000
_BATCH = 512
_LANES = 16


def _body(xt_hbm, out_hbm, xv, ov, pbuf, p8, shared, sem, zsem):
    c = lax.axis_index("c")
    s = lax.axis_index("s")
    g = s // 8
    q = s - g * 8
    bj = 2 * c + g
    b0 = bj * 128

    cp = pltpu.async_copy(
        xt_hbm.at[:, q, pl.ds(0, 8), pl.ds(b0, 128)], xv, sem
    )

    zeros16 = jnp.zeros((_LANES,), jnp.float32)

    @pl.when(s == 0)
    def _mk_template():
        def _zrow(row, carry):
            for j in range(8):
                p8[row, pl.ds(j * _LANES, _LANES)] = zeros16
            return carry

        lax.fori_loop(0, 8, _zrow, 0)
        pltpu.sync_copy(p8, shared.at[pl.ds(16, 8), :])

    plsc.subcore_barrier()

    zcps = [
        pltpu.async_copy(
            shared.at[pl.ds(16, 8), :], ov.at[pl.ds(j * 8, 8), :], zsem
        )
        for j in range(16)
    ]

    cp.wait()

    def _red_k(k, carry):
        koff = pl.multiple_of(k * _LANES, _LANES)

        def _red_c(ch, acc):
            def _red_w(w, a):
                return a + xv[ch, w, pl.ds(koff, _LANES)]

            return lax.fori_loop(0, 8, _red_w, acc)

        acc = lax.fori_loop(0, 3, _red_c, zeros16)
        pbuf[pl.ds(koff, _LANES)] = acc
        return carry

    lax.fori_loop(0, 8, _red_k, 0)

    pltpu.sync_copy(pbuf, shared.at[s])
    plsc.subcore_barrier()
    pltpu.sync_copy(shared.at[pl.ds(g * 8, 8), :], p8)

    for zcp in zcps:
        zcp.wait()

    rowbase = q * 128
    lanes = lax.iota(jnp.int32, _LANES)
    tens = jnp.full((_LANES,), 10.0, jnp.float32)

    def _scat_k(k, carry):
        koff = pl.multiple_of(k * _LANES, _LANES)

        def _comb(i, tot):
            return tot + p8[i, pl.ds(koff, _LANES)]

        total = lax.fori_loop(0, 8, _comb, zeros16)
        mean = total / 192.0
        scaled = mean * 10.0
        pred = scaled.astype(jnp.int32)
        rem = lax.rem(pred, _NUM_CLASSES)
        pred = jnp.where(rem < 0, rem + _NUM_CLASSES, rem)

        r = pred - rowbase
        mask = (r >= 0) & (r < 128)
        r = jnp.where(mask, r, 0)
        plsc.store_scatter(ov, [r, lanes + k * _LANES], tens, mask=mask)
        return carry

    lax.fori_loop(0, 8, _scat_k, 0)

    @pl.when(q < 7)
    def _full():
        pltpu.sync_copy(ov, out_hbm.at[pl.ds(rowbase, 128), pl.ds(b0, 128)])

    @pl.when(q == 7)
    def _tail():
        pltpu.sync_copy(
            ov.at[pl.ds(0, _NUM_CLASSES - 7 * 128), :],
            out_hbm.at[pl.ds(rowbase, _NUM_CLASSES - 7 * 128), pl.ds(b0, 128)],
        )


@jax.jit
def kernel(x):
    xt = jnp.transpose(x, (1, 2, 3, 0))
    out_t = pl.kernel(
        _body,
        out_type=jax.ShapeDtypeStruct((_NUM_CLASSES, _BATCH), jnp.float32),
        mesh=plsc.VectorSubcoreMesh(core_axis_name="c", subcore_axis_name="s"),
        scratch_types=[
            pltpu.VMEM((3, 8, 128), jnp.float32),
            pltpu.VMEM((128, 128), jnp.float32),
            pltpu.VMEM((128,), jnp.float32),
            pltpu.VMEM((8, 128), jnp.float32),
            pltpu.VMEM_SHARED((24, 128), jnp.float32),
            pltpu.SemaphoreType.DMA,
            pltpu.SemaphoreType.DMA,
        ],
        compiler_params=pltpu.CompilerParams(needs_layout_passes=False),
    )(xt)
    return out_t.T
